# baseline (speedup 1.0000x reference)
import functools

import jax
import jax.numpy as jnp
from jax import lax
from jax.experimental import pallas as pl
from jax.experimental.pallas import tpu as pltpu
from jax.experimental.pallas import tpu_sc as plsc

N = 10000
E = 160000
C = 256
CH = 128
NS = 16
K = 128
BLK = 8
CPT = 80
NBLK = CPT // BLK
EPT = K * CPT
EPAD = NS * EPT
NPAD = 10112
ZR = NPAD // NS
NB = 2000
GRID = N // NB


def _mm_body(x_ref, wvt_ref, wnt_ref, fv_ref, fn_ref):
    x = x_ref[...]
    fv = lax.dot_general(x, wvt_ref[...], (((0,), (0,)), ((), ())),
                         preferred_element_type=jnp.float32)
    fn = lax.dot_general(x, wnt_ref[...], (((0,), (0,)), ((), ())),
                         preferred_element_type=jnp.float32)
    fv_ref[...] = fv
    fn_ref[0] = fn[:, :CH]
    fn_ref[1] = fn[:, CH:]


def _tc_matmuls(x, wvt, wnt):
    return pl.pallas_call(
        _mm_body,
        out_shape=[
            jax.ShapeDtypeStruct((N, C), jnp.float32),
            jax.ShapeDtypeStruct((2, N, CH), jnp.float32),
        ],
    )(x, wvt, wnt)


def _sc_body(fn_hbm, gr_hbm, za_hbm, on_hbm,
             sums_o, cnt_o, acc, gr_blk, rows, gsem, ssem):
    c = lax.axis_index("c")
    s = lax.axis_index("s")

    pltpu.sync_copy(za_hbm, acc.at[pl.ds(s * ZR, ZR)])
    plsc.subcore_barrier()

    def gather(k, br):
        pltpu.async_copy(fn_hbm.at[gr_blk.at[k]], rows.at[br], gsem)

    def gather_wait(k, br):
        pltpu.make_async_copy(fn_hbm.at[gr_blk.at[k]], rows.at[br],
                              gsem).wait()

    def scatter(k, br):
        pltpu.async_copy(rows.at[br], acc.at[gr_blk.at[BLK + k]], ssem,
                         add=True)

    def scatter_wait(k, br):
        pltpu.make_async_copy(rows.at[br], acc.at[gr_blk.at[BLK + k]],
                              ssem).wait()

    def block_body(b, last):
        for k in range(BLK):
            gather_wait(k, k % 2)
            if k < BLK - 1:
                if k >= 1:
                    scatter_wait(k - 1, (k + 1) % 2)
                gather(k + 1, (k + 1) % 2)
            scatter(k, k % 2)
        scatter_wait(BLK - 1, (BLK - 1) % 2)
        if not last:
            pltpu.sync_copy(gr_hbm.at[c, s, b + 1], gr_blk)
            gather(0, 0)

    pltpu.sync_copy(gr_hbm.at[c, s, 0], gr_blk)
    gather(0, 0)

    @pl.loop(0, NBLK - 1)
    def _(b):
        block_body(b, False)

    block_body(NBLK - 1, True)

    plsc.subcore_barrier()
    pltpu.sync_copy(acc.at[pl.ds(s * ZR, ZR)],
                    sums_o.at[c, pl.ds(s * ZR, ZR)])
    pltpu.sync_copy(za_hbm, acc.at[pl.ds(s * ZR, ZR)])
    pltpu.sync_copy(on_hbm, rows.at[0])
    plsc.subcore_barrier()

    CBLK = NBLK // 2

    def cblock_body(b, last):
        for k in range(BLK):
            pltpu.async_copy(rows.at[0], acc.at[gr_blk.at[BLK + k]], ssem,
                             add=True)
        for k in range(BLK):
            pltpu.make_async_copy(rows.at[0], acc.at[gr_blk.at[BLK + k]],
                                  ssem).wait()
        if not last:
            pltpu.sync_copy(gr_hbm.at[c, s, c * CBLK + b + 1], gr_blk)

    pltpu.sync_copy(gr_hbm.at[c, s, c * CBLK], gr_blk)

    @pl.loop(0, CBLK - 1)
    def _(b):
        cblock_body(b, False)

    cblock_body(CBLK - 1, True)

    plsc.subcore_barrier()
    pltpu.sync_copy(acc.at[pl.ds(s * ZR, ZR)],
                    cnt_o.at[c, pl.ds(s * ZR, ZR)])


def _sc_segment(fn_cat, gridx, za, ones_h):
    mesh = plsc.VectorSubcoreMesh(core_axis_name="c", subcore_axis_name="s")
    run = pl.kernel(
        _sc_body,
        out_type=(
            jax.ShapeDtypeStruct((2, NPAD, CH), jnp.float32),
            jax.ShapeDtypeStruct((2, NPAD, CH), jnp.float32),
        ),
        mesh=mesh,
        scratch_types=[
            pltpu.VMEM_SHARED((NPAD, CH), jnp.float32),
            pltpu.VMEM((2 * BLK, K), jnp.int32),
            pltpu.VMEM((2, K, CH), jnp.float32),
            pltpu.SemaphoreType.DMA,
            pltpu.SemaphoreType.DMA,
        ],
    )
    return run(fn_cat, gridx, za, ones_h)


def _c1_body(sums_ref, cnt_ref, fv_ref, o_ref, stats_ref):
    s = jnp.concatenate([sums_ref[0][:N], sums_ref[1][:N]], axis=1)
    cnt = cnt_ref[0][:N, 0:1] + cnt_ref[1][:N, 0:1]
    mean = s / jnp.maximum(cnt, 1.0)
    o = mean + fv_ref[...]
    o_ref[...] = o
    stats_ref[0:1, :] = jnp.sum(o, axis=0, keepdims=True)
    stats_ref[1:2, :] = jnp.sum(o * o, axis=0, keepdims=True)
    stats_ref[2:8, :] = jnp.zeros((6, C), jnp.float32)


def _tc_mean_stats(sums, counts, fv):
    return pl.pallas_call(
        _c1_body,
        out_shape=[
            jax.ShapeDtypeStruct((N, C), jnp.float32),
            jax.ShapeDtypeStruct((8, C), jnp.float32),
        ],
    )(sums, counts, fv)


def _c2_body(o_ref, stats_ref, g_ref, b_ref, a_ref, out_ref):
    o = o_ref[...]
    mu = stats_ref[0:1, :] / N
    var = stats_ref[1:2, :] / N - mu * mu
    inv = lax.rsqrt(var + 1e-5)
    y = (o - mu) * (inv * g_ref[...]) + b_ref[...]
    y = jnp.where(y > 0, y, a_ref[0, 0] * y)
    out_ref[...] = y.T


def _tc_norm(o, stats, gamma, beta, alpha):
    return pl.pallas_call(
        _c2_body,
        out_shape=jax.ShapeDtypeStruct((C, N), jnp.float32),
    )(o, stats, gamma.reshape(1, C), beta.reshape(1, C),
      alpha.reshape(1, 1))


def kernel(in_features, reduce_index, gather_index, W_v, W_n, gamma, beta, alpha):
    x = in_features.reshape(C, N)
    fv, fn = _tc_matmuls(x, W_v.T, W_n.T)
    fn_cat = fn.reshape(2 * N, CH)

    pad = EPAD - E
    g32 = gather_index.astype(jnp.int32)
    r32 = reduce_index.astype(jnp.int32)
    gp = jnp.concatenate([g32, jnp.zeros((pad,), jnp.int32)])
    rp = jnp.concatenate([r32, jnp.full((pad,), N, jnp.int32)])
    g4 = gp.reshape(NS, NBLK, BLK, K)
    r4 = rp.reshape(NS, NBLK, BLK, K)
    gridx = jnp.stack([
        jnp.concatenate([g4, r4], axis=2),
        jnp.concatenate([g4 + N, r4], axis=2),
    ])

    za = jnp.zeros((ZR, CH), jnp.float32)
    ones_h = jnp.ones((K, CH), jnp.float32)

    sums, counts = _sc_segment(fn_cat, gridx, za, ones_h)
    o, stats = _tc_mean_stats(sums, counts, fv)
    out = _tc_norm(o, stats, gamma, beta, alpha)
    return out[None]


# speedup vs baseline: 1.1433x; 1.1433x over previous
import functools

import jax
import jax.numpy as jnp
from jax import lax
from jax.experimental import pallas as pl
from jax.experimental.pallas import tpu as pltpu
from jax.experimental.pallas import tpu_sc as plsc

N = 10000
E = 160000
C = 256
CH = 128
NS = 16
K = 64
CPT = 160
EPT = K * CPT
EPAD = NS * EPT
NPAD = 10240
ZR = NPAD // NS
NB = 2000
GRID = N // NB


def _mm_body(x_ref, wvt_ref, wnt_ref, fv_ref, fn_ref):
    x = x_ref[...]
    fv = lax.dot_general(x, wvt_ref[...], (((0,), (0,)), ((), ())),
                         preferred_element_type=jnp.float32)
    fn = lax.dot_general(x, wnt_ref[...], (((0,), (0,)), ((), ())),
                         preferred_element_type=jnp.float32)
    fv_ref[...] = fv
    fn_ref[0] = fn[:, :CH]
    fn_ref[1] = fn[:, CH:]


def _tc_matmuls(x, wvt, wnt):
    return pl.pallas_call(
        _mm_body,
        out_shape=[
            jax.ShapeDtypeStruct((N, C), jnp.float32),
            jax.ShapeDtypeStruct((2, N, CH), jnp.float32),
        ],
    )(x, wvt, wnt)


def _sc_body(fn_hbm, g_hbm, r_hbm, za_hbm, on_hbm,
             sums_o, cnt_o, acc, g_v, r_v, rows, gsem, isem, ssem):
    c = lax.axis_index("c")
    s = lax.axis_index("s")

    pltpu.sync_copy(za_hbm, acc.at[pl.ds(s * ZR, ZR)])
    plsc.subcore_barrier()

    gbase = (c * NS + s) * EPT
    rbase = s * EPT

    def idx_load_async(j, b):
        pltpu.async_copy(g_hbm.at[pl.ds(gbase + j * K, K)], g_v.at[b], isem)
        pltpu.async_copy(r_hbm.at[pl.ds(rbase + j * K, K)], r_v.at[b], isem)

    def idx_wait(j, b):
        pltpu.make_async_copy(g_hbm.at[pl.ds(gbase + j * K, K)], g_v.at[b],
                              isem).wait()
        pltpu.make_async_copy(r_hbm.at[pl.ds(rbase + j * K, K)], r_v.at[b],
                              isem).wait()

    def gather(br, bi):
        pltpu.async_copy(fn_hbm.at[g_v.at[bi]], rows.at[br], gsem)

    def gather_wait(br, bi):
        pltpu.make_async_copy(fn_hbm.at[g_v.at[bi]], rows.at[br], gsem).wait()

    def scatter(br, bi):
        pltpu.async_copy(rows.at[br], acc.at[r_v.at[bi]], ssem, add=True)

    def scatter_wait(br, bi):
        pltpu.make_async_copy(rows.at[br], acc.at[r_v.at[bi]], ssem).wait()

    def step(j, wait_sc, fire_g, fire_i):
        if wait_sc:
            scatter_wait(lax.rem(j + 3, 4), lax.rem(j + 4, 5))
        if fire_g:
            idx_wait(j + 3, lax.rem(j + 3, 5))
            gather(lax.rem(j + 3, 4), lax.rem(j + 3, 5))
        gather_wait(lax.rem(j, 4), lax.rem(j, 5))
        scatter(lax.rem(j, 4), lax.rem(j, 5))
        if fire_i:
            idx_load_async(j + 4, lax.rem(j + 4, 5))

    for p in range(3):
        pltpu.sync_copy(g_hbm.at[pl.ds(gbase + p * K, K)], g_v.at[p])
        pltpu.sync_copy(r_hbm.at[pl.ds(rbase + p * K, K)], r_v.at[p])
        gather(p, p)
    idx_load_async(3, 3)

    step(0, False, True, True)

    @pl.loop(1, CPT - 4)
    def _(j):
        step(j, True, True, True)

    step(CPT - 4, True, True, False)
    step(CPT - 3, True, False, False)
    step(CPT - 2, True, False, False)
    step(CPT - 1, True, False, False)
    scatter_wait((CPT - 1) % 4, (CPT - 1) % 5)

    plsc.subcore_barrier()
    pltpu.sync_copy(acc.at[pl.ds(s * ZR, ZR)],
                    sums_o.at[c, pl.ds(s * ZR, ZR)])
    pltpu.sync_copy(za_hbm, acc.at[pl.ds(s * ZR, ZR)])
    pltpu.sync_copy(on_hbm, rows.at[0])
    plsc.subcore_barrier()

    CCPT = CPT // 2
    cbase = (c * NS + s) * (EPT // 2)

    def cidx_load_async(j, b):
        pltpu.async_copy(r_hbm.at[pl.ds(cbase + j * K, K)], r_v.at[b], isem)

    def cidx_wait(j, b):
        pltpu.make_async_copy(r_hbm.at[pl.ds(cbase + j * K, K)], r_v.at[b],
                              isem).wait()

    def cscatter(b):
        pltpu.async_copy(rows.at[0], acc.at[r_v.at[b]], ssem, add=True)

    def cscatter_wait(b):
        pltpu.make_async_copy(rows.at[0], acc.at[r_v.at[b]], ssem).wait()

    def cstep(j, first, last2, last1):
        if not last1:
            cidx_wait(j + 1, lax.rem(j + 1, 4))
        if not first:
            cscatter_wait(lax.rem(j + 2, 4))
        cscatter(lax.rem(j, 4))
        if not (last1 or last2):
            cidx_load_async(j + 2, lax.rem(j + 2, 4))

    pltpu.sync_copy(r_hbm.at[pl.ds(cbase, K)], r_v.at[0])
    cidx_load_async(1, 1)

    cstep(0, True, False, False)
    cstep(1, True, False, False)

    @pl.loop(2, CCPT - 2)
    def _(j):
        cstep(j, False, False, False)

    cstep(CCPT - 2, False, True, False)
    cstep(CCPT - 1, False, True, True)
    cscatter_wait((CCPT - 2) % 4)
    cscatter_wait((CCPT - 1) % 4)

    plsc.subcore_barrier()
    pltpu.sync_copy(acc.at[pl.ds(s * ZR, ZR)],
                    cnt_o.at[c, pl.ds(s * ZR, ZR)])


def _sc_segment(fn_cat, gidx, ridx, za, ones_h):
    mesh = plsc.VectorSubcoreMesh(core_axis_name="c", subcore_axis_name="s")
    run = pl.kernel(
        _sc_body,
        out_type=(
            jax.ShapeDtypeStruct((2, NPAD, CH), jnp.float32),
            jax.ShapeDtypeStruct((2, NPAD, CH), jnp.float32),
        ),
        mesh=mesh,
        scratch_types=[
            pltpu.VMEM_SHARED((NPAD, CH), jnp.float32),
            pltpu.VMEM((5, K), jnp.int32),
            pltpu.VMEM((5, K), jnp.int32),
            pltpu.VMEM((4, K, CH), jnp.float32),
            pltpu.SemaphoreType.DMA,
            pltpu.SemaphoreType.DMA,
            pltpu.SemaphoreType.DMA,
        ],
    )
    return run(fn_cat, gidx, ridx, za, ones_h)


def _c1_body(sums_ref, cnt_ref, fv_ref, o_ref, stats_ref):
    s = jnp.concatenate([sums_ref[0][:N], sums_ref[1][:N]], axis=1)
    cnt = cnt_ref[0][:N, 0:1] + cnt_ref[1][:N, 0:1]
    mean = s / jnp.maximum(cnt, 1.0)
    o = mean + fv_ref[...]
    o_ref[...] = o
    stats_ref[0:1, :] = jnp.sum(o, axis=0, keepdims=True)
    stats_ref[1:2, :] = jnp.sum(o * o, axis=0, keepdims=True)
    stats_ref[2:8, :] = jnp.zeros((6, C), jnp.float32)


def _tc_mean_stats(sums, counts, fv):
    return pl.pallas_call(
        _c1_body,
        out_shape=[
            jax.ShapeDtypeStruct((N, C), jnp.float32),
            jax.ShapeDtypeStruct((8, C), jnp.float32),
        ],
    )(sums, counts, fv)


def _c2_body(o_ref, stats_ref, g_ref, b_ref, a_ref, out_ref):
    o = o_ref[...]
    mu = stats_ref[0:1, :] / N
    var = stats_ref[1:2, :] / N - mu * mu
    inv = lax.rsqrt(var + 1e-5)
    y = (o - mu) * (inv * g_ref[...]) + b_ref[...]
    y = jnp.where(y > 0, y, a_ref[0, 0] * y)
    out_ref[...] = y.T


def _tc_norm(o, stats, gamma, beta, alpha):
    return pl.pallas_call(
        _c2_body,
        out_shape=jax.ShapeDtypeStruct((C, N), jnp.float32),
    )(o, stats, gamma.reshape(1, C), beta.reshape(1, C),
      alpha.reshape(1, 1))


def kernel(in_features, reduce_index, gather_index, W_v, W_n, gamma, beta, alpha):
    x = in_features.reshape(C, N)
    fv, fn = _tc_matmuls(x, W_v.T, W_n.T)
    fn_cat = fn.reshape(2 * N, CH)

    pad = EPAD - E
    g32 = gather_index.astype(jnp.int32)
    r32 = reduce_index.astype(jnp.int32)
    gp = jnp.concatenate([g32, jnp.zeros((pad,), jnp.int32)])
    rp = jnp.concatenate([r32, jnp.full((pad,), N, jnp.int32)])
    gidx = jnp.stack([gp, gp + N]).reshape(2 * EPAD)
    ridx = rp

    za = jnp.zeros((ZR, CH), jnp.float32)
    ones_h = jnp.ones((K, CH), jnp.float32)

    sums, counts = _sc_segment(fn_cat, gidx, ridx, za, ones_h)
    o, stats = _tc_mean_stats(sums, counts, fv)
    out = _tc_norm(o, stats, gamma, beta, alpha)
    return out[None]
